# initial kernel scaffold (unmeasured)
import jax
import jax.numpy as jnp
from jax import lax
from jax.experimental import pallas as pl
from jax.experimental.pallas import tpu as pltpu


def kernel(
    x,
):
    def body(*refs):
        pass

    out_shape = jax.ShapeDtypeStruct(..., jnp.float32)
    return pl.pallas_call(body, out_shape=out_shape)(...)


# baseline (device time: 96556 ns/iter reference)
import jax
import jax.numpy as jnp
from jax import lax
from jax.experimental import pallas as pl
from jax.experimental.pallas import tpu as pltpu

M = 1024
N = 1024


def kernel(x):
    def body(x_ref, out_ref, recv_ref, send_sems, recv_sems):
        mx = lax.axis_index("x")
        my = lax.axis_index("y")
        mz = lax.axis_index("z")

        barrier = pltpu.get_barrier_semaphore()
        for nbr in ((1 - mx, my, mz), (mx, 1 - my, mz), (mx, my, 1 - mz)):
            pl.semaphore_signal(
                barrier, inc=1, device_id=nbr,
                device_id_type=pl.DeviceIdType.MESH,
            )
        pl.semaphore_wait(barrier, 3)

        out_ref[:, :] = x_ref[0, 0, 0, :, :]

        def rs_step(i, partner, keep_base, send_base, rows):
            rdma = pltpu.make_async_remote_copy(
                src_ref=out_ref.at[pl.ds(send_base, rows)],
                dst_ref=recv_ref.at[pl.ds(0, rows)],
                send_sem=send_sems.at[i],
                recv_sem=recv_sems.at[i],
                device_id=partner,
                device_id_type=pl.DeviceIdType.MESH,
            )
            rdma.start()
            rdma.wait()
            out_ref[pl.ds(keep_base, rows), :] = (
                out_ref[pl.ds(keep_base, rows), :]
                + recv_ref[pl.ds(0, rows), :]
            )

        def ag_step(i, partner, base, rows):
            rdma = pltpu.make_async_remote_copy(
                src_ref=out_ref.at[pl.ds(base, rows)],
                dst_ref=out_ref.at[pl.ds(base, rows)],
                send_sem=send_sems.at[i],
                recv_sem=recv_sems.at[i],
                device_id=partner,
                device_id_type=pl.DeviceIdType.MESH,
            )
            rdma.start()
            rdma.wait()

        px = (1 - mx, my, mz)
        py = (mx, 1 - my, mz)
        pz = (mx, my, 1 - mz)

        rs_step(0, px, mx * 512, (1 - mx) * 512, 512)
        rs_step(1, py, mx * 512 + my * 256, mx * 512 + (1 - my) * 256, 256)
        rs_step(
            2, pz,
            mx * 512 + my * 256 + mz * 128,
            mx * 512 + my * 256 + (1 - mz) * 128,
            128,
        )

        ag_step(3, pz, mx * 512 + my * 256 + mz * 128, 128)
        ag_step(4, py, mx * 512 + my * 256, 256)
        ag_step(5, px, mx * 512, 512)

    return pl.pallas_call(
        body,
        out_shape=jax.ShapeDtypeStruct((M, N), jnp.float32),
        in_specs=[pl.BlockSpec(memory_space=pltpu.VMEM)],
        out_specs=pl.BlockSpec(memory_space=pltpu.VMEM),
        scratch_shapes=[
            pltpu.VMEM((512, N), jnp.float32),
            pltpu.SemaphoreType.DMA((6,)),
            pltpu.SemaphoreType.DMA((6,)),
        ],
        compiler_params=pltpu.CompilerParams(collective_id=0),
    )(x)


# device time: 47378 ns/iter; 2.0380x vs baseline; 2.0380x over previous
import jax
import jax.numpy as jnp
from jax import lax
from jax.experimental import pallas as pl
from jax.experimental.pallas import tpu as pltpu

M = 1024
N = 1024

_CHUNKS = (
    (0, 384, ("x", "y", "z")),
    (384, 320, ("y", "z", "x")),
    (704, 320, ("z", "x", "y")),
)
_N_STEPS = 6
_STAGE_ROWS = 336


def kernel(x):
    def body(x_ref, out_ref, recv_ref, send_sems, recv_sems):
        bits = {
            "x": lax.axis_index("x"),
            "y": lax.axis_index("y"),
            "z": lax.axis_index("z"),
        }
        partners = {
            "x": (1 - bits["x"], bits["y"], bits["z"]),
            "y": (bits["x"], 1 - bits["y"], bits["z"]),
            "z": (bits["x"], bits["y"], 1 - bits["z"]),
        }

        barrier = pltpu.get_barrier_semaphore()
        for ax in ("x", "y", "z"):
            pl.semaphore_signal(
                barrier, inc=1, device_id=partners[ax],
                device_id_type=pl.DeviceIdType.MESH,
            )
        pl.semaphore_wait(barrier, 3)

        out_ref[:, :] = x_ref[0, 0, 0, :, :]

        steps = []
        for base, w, order in _CHUNKS:
            sizes = (w // 2, w // 4, w // 8)
            prefixes = [base]
            for k in range(3):
                prefixes.append(prefixes[-1] + bits[order[k]] * sizes[k])
            offs = (0, sizes[0], sizes[0] + sizes[1])
            plan = []
            for k in range(3):
                send = prefixes[k] + (1 - bits[order[k]]) * sizes[k]
                plan.append(("rs", order[k], sizes[k], send, prefixes[k + 1], offs[k]))
            for k in (2, 1, 0):
                plan.append(("ag", order[k], sizes[k], prefixes[k + 1], None, None))
            steps.append(plan)

        def make_rdma(c, i):
            kind, ax, rows, sbase, _keep, off = steps[c][i]
            src = out_ref.at[pl.ds(sbase, rows)]
            if kind == "rs":
                dst = recv_ref.at[c, pl.ds(off, rows)]
            else:
                dst = out_ref.at[pl.ds(sbase, rows)]
            return pltpu.make_async_remote_copy(
                src_ref=src,
                dst_ref=dst,
                send_sem=send_sems.at[c, i],
                recv_sem=recv_sems.at[c, i],
                device_id=partners[ax],
                device_id_type=pl.DeviceIdType.MESH,
            )

        for c in range(len(_CHUNKS)):
            make_rdma(c, 0).start()
        for i in range(_N_STEPS):
            for c in range(len(_CHUNKS)):
                make_rdma(c, i).wait()
                kind, _ax, rows, _sbase, keep, off = steps[c][i]
                if kind == "rs":
                    out_ref[pl.ds(keep, rows), :] = (
                        out_ref[pl.ds(keep, rows), :]
                        + recv_ref[c, pl.ds(off, rows), :]
                    )
                if i + 1 < _N_STEPS:
                    make_rdma(c, i + 1).start()

    return pl.pallas_call(
        body,
        out_shape=jax.ShapeDtypeStruct((M, N), jnp.float32),
        in_specs=[pl.BlockSpec(memory_space=pltpu.VMEM)],
        out_specs=pl.BlockSpec(memory_space=pltpu.VMEM),
        scratch_shapes=[
            pltpu.VMEM((3, _STAGE_ROWS, N), jnp.float32),
            pltpu.SemaphoreType.DMA((3, _N_STEPS)),
            pltpu.SemaphoreType.DMA((3, _N_STEPS)),
        ],
        compiler_params=pltpu.CompilerParams(collective_id=0),
    )(x)


# device time: 40090 ns/iter; 2.4085x vs baseline; 1.1818x over previous
import jax
import jax.numpy as jnp
from jax import lax
from jax.experimental import pallas as pl
from jax.experimental.pallas import tpu as pltpu

M = 1024
N = 1024

_CHUNKS = (
    (0, 192, ("x", "y", "z")),
    (384, 192, ("y", "z", "x")),
    (704, 192, ("z", "x", "y")),
    (192, 192, ("x", "y", "z")),
    (576, 128, ("y", "z", "x")),
    (896, 128, ("z", "x", "y")),
)
_N_CHUNKS = len(_CHUNKS)
_N_STEPS = 6
_STAGE_ROWS = 168


def kernel(x):
    def body(x_ref, out_ref, recv_ref, send_sems, recv_sems):
        bits = {
            "x": lax.axis_index("x"),
            "y": lax.axis_index("y"),
            "z": lax.axis_index("z"),
        }
        partners = {
            "x": (1 - bits["x"], bits["y"], bits["z"]),
            "y": (bits["x"], 1 - bits["y"], bits["z"]),
            "z": (bits["x"], bits["y"], 1 - bits["z"]),
        }

        barrier = pltpu.get_barrier_semaphore()
        for ax in ("x", "y", "z"):
            pl.semaphore_signal(
                barrier, inc=1, device_id=partners[ax],
                device_id_type=pl.DeviceIdType.MESH,
            )
        pl.semaphore_wait(barrier, 3)

        steps = []
        for base, w, order in _CHUNKS:
            sizes = (w // 2, w // 4, w // 8)
            prefixes = [base]
            for k in range(3):
                prefixes.append(prefixes[-1] + bits[order[k]] * sizes[k])
            offs = (0, sizes[0], sizes[0] + sizes[1])
            plan = []
            for k in range(3):
                send = prefixes[k] + (1 - bits[order[k]]) * sizes[k]
                plan.append(("rs", order[k], sizes[k], send, prefixes[k + 1], offs[k]))
            for k in (2, 1, 0):
                plan.append(("ag", order[k], sizes[k], prefixes[k + 1], None, None))
            steps.append(plan)

        def make_rdma(c, i):
            kind, ax, rows, sbase, _keep, off = steps[c][i]
            if i == 0:
                src = x_ref.at[0, 0, 0, pl.ds(sbase, rows)]
            else:
                src = out_ref.at[pl.ds(sbase, rows)]
            if kind == "rs":
                dst = recv_ref.at[c, pl.ds(off, rows)]
            else:
                dst = out_ref.at[pl.ds(sbase, rows)]
            return pltpu.make_async_remote_copy(
                src_ref=src,
                dst_ref=dst,
                send_sem=send_sems.at[c, i],
                recv_sem=recv_sems.at[c, i],
                device_id=partners[ax],
                device_id_type=pl.DeviceIdType.MESH,
            )

        for c in range(_N_CHUNKS):
            make_rdma(c, 0).start()
        for i in range(_N_STEPS):
            for c in range(_N_CHUNKS):
                make_rdma(c, i).wait_recv()
                kind, _ax, rows, _sbase, keep, off = steps[c][i]
                if kind == "rs":
                    own = (
                        x_ref[0, 0, 0, pl.ds(keep, rows), :]
                        if i == 0
                        else out_ref[pl.ds(keep, rows), :]
                    )
                    out_ref[pl.ds(keep, rows), :] = (
                        own + recv_ref[c, pl.ds(off, rows), :]
                    )
                if i + 1 < _N_STEPS:
                    make_rdma(c, i + 1).start()
        for i in range(_N_STEPS):
            for c in range(_N_CHUNKS):
                make_rdma(c, i).wait_send()

    return pl.pallas_call(
        body,
        out_shape=jax.ShapeDtypeStruct((M, N), jnp.float32),
        in_specs=[pl.BlockSpec(memory_space=pltpu.VMEM)],
        out_specs=pl.BlockSpec(memory_space=pltpu.VMEM),
        scratch_shapes=[
            pltpu.VMEM((_N_CHUNKS, _STAGE_ROWS, N), jnp.float32),
            pltpu.SemaphoreType.DMA((_N_CHUNKS, _N_STEPS)),
            pltpu.SemaphoreType.DMA((_N_CHUNKS, _N_STEPS)),
        ],
        compiler_params=pltpu.CompilerParams(collective_id=0),
    )(x)


# device time: 39067 ns/iter; 2.4715x vs baseline; 1.0262x over previous
import jax
import jax.numpy as jnp
from jax import lax
from jax.experimental import pallas as pl
from jax.experimental.pallas import tpu as pltpu

M = 1024
N = 1024

_CHUNKS = (
    (0, 128, ("x", "y", "z")),
    (384, 128, ("y", "z", "x")),
    (704, 128, ("z", "x", "y")),
    (128, 128, ("x", "y", "z")),
    (512, 128, ("y", "z", "x")),
    (832, 128, ("z", "x", "y")),
    (256, 128, ("x", "y", "z")),
    (640, 64, ("y", "z", "x")),
    (960, 64, ("z", "x", "y")),
)
_N_CHUNKS = len(_CHUNKS)
_N_STEPS = 6
_STAGE_ROWS = 112


def kernel(x):
    def body(x_ref, out_ref, recv_ref, send_sems, recv_sems):
        bits = {
            "x": lax.axis_index("x"),
            "y": lax.axis_index("y"),
            "z": lax.axis_index("z"),
        }
        partners = {
            "x": (1 - bits["x"], bits["y"], bits["z"]),
            "y": (bits["x"], 1 - bits["y"], bits["z"]),
            "z": (bits["x"], bits["y"], 1 - bits["z"]),
        }

        barrier = pltpu.get_barrier_semaphore()
        for ax in ("x", "y", "z"):
            pl.semaphore_signal(
                barrier, inc=1, device_id=partners[ax],
                device_id_type=pl.DeviceIdType.MESH,
            )
        pl.semaphore_wait(barrier, 3)

        steps = []
        for base, w, order in _CHUNKS:
            sizes = (w // 2, w // 4, w // 8)
            prefixes = [base]
            for k in range(3):
                prefixes.append(prefixes[-1] + bits[order[k]] * sizes[k])
            offs = (0, sizes[0], sizes[0] + sizes[1])
            plan = []
            for k in range(3):
                send = prefixes[k] + (1 - bits[order[k]]) * sizes[k]
                plan.append(("rs", order[k], sizes[k], send, prefixes[k + 1], offs[k]))
            for k in (2, 1, 0):
                plan.append(("ag", order[k], sizes[k], prefixes[k + 1], None, None))
            steps.append(plan)

        def make_rdma(c, i):
            kind, ax, rows, sbase, _keep, off = steps[c][i]
            if i == 0:
                src = x_ref.at[0, 0, 0, pl.ds(sbase, rows)]
            else:
                src = out_ref.at[pl.ds(sbase, rows)]
            if kind == "rs":
                dst = recv_ref.at[c, pl.ds(off, rows)]
            else:
                dst = out_ref.at[pl.ds(sbase, rows)]
            return pltpu.make_async_remote_copy(
                src_ref=src,
                dst_ref=dst,
                send_sem=send_sems.at[c, i],
                recv_sem=recv_sems.at[c, i],
                device_id=partners[ax],
                device_id_type=pl.DeviceIdType.MESH,
            )

        for c in range(_N_CHUNKS):
            make_rdma(c, 0).start()
        for i in range(_N_STEPS):
            for c in range(_N_CHUNKS):
                make_rdma(c, i).wait_recv()
                kind, _ax, rows, _sbase, keep, off = steps[c][i]
                if kind == "rs":
                    own = (
                        x_ref[0, 0, 0, pl.ds(keep, rows), :]
                        if i == 0
                        else out_ref[pl.ds(keep, rows), :]
                    )
                    out_ref[pl.ds(keep, rows), :] = (
                        own + recv_ref[c, pl.ds(off, rows), :]
                    )
                if i + 1 < _N_STEPS:
                    make_rdma(c, i + 1).start()
        for i in range(_N_STEPS):
            for c in range(_N_CHUNKS):
                make_rdma(c, i).wait_send()

    return pl.pallas_call(
        body,
        out_shape=jax.ShapeDtypeStruct((M, N), jnp.float32),
        in_specs=[pl.BlockSpec(memory_space=pltpu.VMEM)],
        out_specs=pl.BlockSpec(memory_space=pltpu.VMEM),
        scratch_shapes=[
            pltpu.VMEM((_N_CHUNKS, _STAGE_ROWS, N), jnp.float32),
            pltpu.SemaphoreType.DMA((_N_CHUNKS, _N_STEPS)),
            pltpu.SemaphoreType.DMA((_N_CHUNKS, _N_STEPS)),
        ],
        compiler_params=pltpu.CompilerParams(collective_id=0),
    )(x)


# device time: 38981 ns/iter; 2.4770x vs baseline; 1.0022x over previous
import jax
import jax.numpy as jnp
from jax import lax
from jax.experimental import pallas as pl
from jax.experimental.pallas import tpu as pltpu

M = 1024
N = 1024

_CHUNKS = (
    (0, 64, ("x", "y", "z")),
    (384, 64, ("y", "z", "x")),
    (704, 64, ("z", "x", "y")),
    (64, 128, ("x", "y", "z")),
    (448, 128, ("y", "z", "x")),
    (768, 128, ("z", "x", "y")),
    (192, 128, ("x", "y", "z")),
    (576, 64, ("y", "z", "x")),
    (896, 64, ("z", "x", "y")),
    (320, 64, ("x", "y", "z")),
    (640, 64, ("y", "z", "x")),
    (960, 64, ("z", "x", "y")),
)
_N_CHUNKS = len(_CHUNKS)
_N_STEPS = 6
_STAGE_ROWS = 112


def kernel(x):
    def body(x_ref, out_ref, recv_ref, send_sems, recv_sems):
        bits = {
            "x": lax.axis_index("x"),
            "y": lax.axis_index("y"),
            "z": lax.axis_index("z"),
        }
        partners = {
            "x": (1 - bits["x"], bits["y"], bits["z"]),
            "y": (bits["x"], 1 - bits["y"], bits["z"]),
            "z": (bits["x"], bits["y"], 1 - bits["z"]),
        }

        barrier = pltpu.get_barrier_semaphore()
        for ax in ("x", "y", "z"):
            pl.semaphore_signal(
                barrier, inc=1, device_id=partners[ax],
                device_id_type=pl.DeviceIdType.MESH,
            )
        pl.semaphore_wait(barrier, 3)

        steps = []
        for base, w, order in _CHUNKS:
            sizes = (w // 2, w // 4, w // 8)
            prefixes = [base]
            for k in range(3):
                prefixes.append(prefixes[-1] + bits[order[k]] * sizes[k])
            offs = (0, sizes[0], sizes[0] + sizes[1])
            plan = []
            for k in range(3):
                send = prefixes[k] + (1 - bits[order[k]]) * sizes[k]
                plan.append(("rs", order[k], sizes[k], send, prefixes[k + 1], offs[k]))
            for k in (2, 1, 0):
                plan.append(("ag", order[k], sizes[k], prefixes[k + 1], None, None))
            steps.append(plan)

        def make_rdma(c, i):
            kind, ax, rows, sbase, _keep, off = steps[c][i]
            if i == 0:
                src = x_ref.at[0, 0, 0, pl.ds(sbase, rows)]
            else:
                src = out_ref.at[pl.ds(sbase, rows)]
            if kind == "rs":
                dst = recv_ref.at[c, pl.ds(off, rows)]
            else:
                dst = out_ref.at[pl.ds(sbase, rows)]
            return pltpu.make_async_remote_copy(
                src_ref=src,
                dst_ref=dst,
                send_sem=send_sems.at[c, i],
                recv_sem=recv_sems.at[c, i],
                device_id=partners[ax],
                device_id_type=pl.DeviceIdType.MESH,
            )

        for c in range(_N_CHUNKS):
            make_rdma(c, 0).start()
        for i in range(_N_STEPS):
            for c in range(_N_CHUNKS):
                make_rdma(c, i).wait_recv()
                kind, _ax, rows, _sbase, keep, off = steps[c][i]
                if kind == "rs":
                    own = (
                        x_ref[0, 0, 0, pl.ds(keep, rows), :]
                        if i == 0
                        else out_ref[pl.ds(keep, rows), :]
                    )
                    out_ref[pl.ds(keep, rows), :] = (
                        own + recv_ref[c, pl.ds(off, rows), :]
                    )
                if i + 1 < _N_STEPS:
                    make_rdma(c, i + 1).start()
        for i in range(_N_STEPS):
            for c in range(_N_CHUNKS):
                make_rdma(c, i).wait_send()

    return pl.pallas_call(
        body,
        out_shape=jax.ShapeDtypeStruct((M, N), jnp.float32),
        in_specs=[pl.BlockSpec(memory_space=pltpu.VMEM)],
        out_specs=pl.BlockSpec(memory_space=pltpu.VMEM),
        scratch_shapes=[
            pltpu.VMEM((_N_CHUNKS, _STAGE_ROWS, N), jnp.float32),
            pltpu.SemaphoreType.DMA((_N_CHUNKS, _N_STEPS)),
            pltpu.SemaphoreType.DMA((_N_CHUNKS, _N_STEPS)),
        ],
        compiler_params=pltpu.CompilerParams(collective_id=0),
    )(x)


# device time: 38818 ns/iter; 2.4874x vs baseline; 1.0042x over previous
import jax
import jax.numpy as jnp
from jax import lax
from jax.experimental import pallas as pl
from jax.experimental.pallas import tpu as pltpu

M = 1024
N = 1024

_CHUNKS = (
    (0, 192, ("x", "y", "z")),
    (384, 192, ("y", "z", "x")),
    (704, 192, ("z", "x", "y")),
    (192, 192, ("x", "y", "z")),
    (576, 128, ("y", "z", "x")),
    (896, 128, ("z", "x", "y")),
)
_N_CHUNKS = len(_CHUNKS)
_N_MSGS = 10
_STAGE_ROWS = 168


def kernel(x):
    def body(x_ref, out_ref, recv_ref, send_sems, recv_sems):
        bits = {
            "x": lax.axis_index("x"),
            "y": lax.axis_index("y"),
            "z": lax.axis_index("z"),
        }
        partners = {
            "x": (1 - bits["x"], bits["y"], bits["z"]),
            "y": (bits["x"], 1 - bits["y"], bits["z"]),
            "z": (bits["x"], bits["y"], 1 - bits["z"]),
        }

        barrier = pltpu.get_barrier_semaphore()
        for ax in ("x", "y", "z"):
            pl.semaphore_signal(
                barrier, inc=1, device_id=partners[ax],
                device_id_type=pl.DeviceIdType.MESH,
            )
        pl.semaphore_wait(barrier, 3)

        rs_plans, ag_plans = [], []
        for base, w, order in _CHUNKS:
            s1, s2, s3 = w // 2, w // 4, w // 8
            b1, b2, b3 = (bits[a] for a in order)
            prefixes = [
                base,
                base + b1 * s1,
                base + b1 * s1 + b2 * s2,
                base + b1 * s1 + b2 * s2 + b3 * s3,
            ]
            offs = (0, s1, s1 + s2)
            rs = []
            for k, s in enumerate((s1, s2, s3)):
                send = prefixes[k] + (1 - (b1, b2, b3)[k]) * s
                rs.append((order[k], s, send, prefixes[k + 1], offs[k]))
            blk = {}
            for i in (0, 1):
                for j in (0, 1):
                    b2e = b2 if i == 0 else 1 - b2
                    b3e = b3 if j == 0 else 1 - b3
                    blk[(i, j)] = base + b1 * s1 + b2e * s2 + b3e * s3
            ag = [
                (order[2], s3, blk[(0, 0)]),
                (order[1], s3, blk[(0, 0)]),
                (order[1], s3, blk[(0, 1)]),
                (order[0], s3, blk[(0, 0)]),
                (order[0], s3, blk[(0, 1)]),
                (order[0], s3, blk[(1, 0)]),
                (order[0], s3, blk[(1, 1)]),
            ]
            rs_plans.append(rs)
            ag_plans.append(ag)

        def rs_rdma(c, k):
            ax, rows, sbase, _keep, off = rs_plans[c][k]
            src = (
                x_ref.at[0, 0, 0, pl.ds(sbase, rows)]
                if k == 0
                else out_ref.at[pl.ds(sbase, rows)]
            )
            return pltpu.make_async_remote_copy(
                src_ref=src,
                dst_ref=recv_ref.at[c, pl.ds(off, rows)],
                send_sem=send_sems.at[c, k],
                recv_sem=recv_sems.at[c, k],
                device_id=partners[ax],
                device_id_type=pl.DeviceIdType.MESH,
            )

        def ag_rdma(c, m):
            ax, rows, bbase = ag_plans[c][m - 3]
            return pltpu.make_async_remote_copy(
                src_ref=out_ref.at[pl.ds(bbase, rows)],
                dst_ref=out_ref.at[pl.ds(bbase, rows)],
                send_sem=send_sems.at[c, m],
                recv_sem=recv_sems.at[c, m],
                device_id=partners[ax],
                device_id_type=pl.DeviceIdType.MESH,
            )

        for c in range(_N_CHUNKS):
            rs_rdma(c, 0).start()
        for k in range(3):
            for c in range(_N_CHUNKS):
                rs_rdma(c, k).wait_recv()
                _ax, rows, _sbase, keep, off = rs_plans[c][k]
                own = (
                    x_ref[0, 0, 0, pl.ds(keep, rows), :]
                    if k == 0
                    else out_ref[pl.ds(keep, rows), :]
                )
                out_ref[pl.ds(keep, rows), :] = (
                    own + recv_ref[c, pl.ds(off, rows), :]
                )
                if k < 2:
                    rs_rdma(c, k + 1).start()
                else:
                    ag_rdma(c, 3).start()
                    ag_rdma(c, 4).start()
                    ag_rdma(c, 6).start()

        for c in range(_N_CHUNKS):
            ag_rdma(c, 3).wait_recv()
            ag_rdma(c, 5).start()
            ag_rdma(c, 7).start()
        for c in range(_N_CHUNKS):
            ag_rdma(c, 4).wait_recv()
            ag_rdma(c, 8).start()
        for c in range(_N_CHUNKS):
            ag_rdma(c, 5).wait_recv()
            ag_rdma(c, 9).start()
        for c in range(_N_CHUNKS):
            for m in (6, 7, 8, 9):
                ag_rdma(c, m).wait_recv()

        for c in range(_N_CHUNKS):
            for k in range(3):
                rs_rdma(c, k).wait_send()
            for m in range(3, _N_MSGS):
                ag_rdma(c, m).wait_send()

    return pl.pallas_call(
        body,
        out_shape=jax.ShapeDtypeStruct((M, N), jnp.float32),
        in_specs=[pl.BlockSpec(memory_space=pltpu.VMEM)],
        out_specs=pl.BlockSpec(memory_space=pltpu.VMEM),
        scratch_shapes=[
            pltpu.VMEM((_N_CHUNKS, _STAGE_ROWS, N), jnp.float32),
            pltpu.SemaphoreType.DMA((_N_CHUNKS, _N_MSGS)),
            pltpu.SemaphoreType.DMA((_N_CHUNKS, _N_MSGS)),
        ],
        compiler_params=pltpu.CompilerParams(collective_id=0),
    )(x)
